# Initial kernel scaffold; baseline (speedup 1.0000x reference)
import jax
import jax.numpy as jnp
from jax.experimental import pallas as pl


def kernel(x, adj, Wq, Wk, Wv, bq, bk, bv, Wo, bo, ln1_g, ln1_b, Wr, br, We1, be1, We2, be2, ln2_g, ln2_b, Wd, bd, Wg, a_src, a_dst):
    raise NotImplementedError("write your pallas kernel here")


# speedup vs baseline: 1.3041x; 1.3041x over previous
import functools

import jax
import jax.numpy as jnp
from jax.experimental import pallas as pl

B, N, L, D = 2, 512, 32, 64
E, K = 16, 2
HT, HN = 4, 4
OUT = 64
DH = D // HT
DHN = OUT // HN
L2 = L // 2
SEQ = B * N

SBLK = 8
TBLK = SBLK * L


def _ln(z, g, b):
    m = jnp.mean(z, axis=-1, keepdims=True)
    v = jnp.mean((z - m) ** 2, axis=-1, keepdims=True)
    return (z - m) / jnp.sqrt(v + 1e-5) * g + b


def _block_kernel(x_ref, wq_ref, wk_ref, wv_ref, bq_ref, bk_ref, bv_ref,
                  wo_ref, bo_ref, ln1g_ref, ln1b_ref, wr_ref, br_ref,
                  we1_ref, be1_ref, we2_ref, be2_ref, ln2g_ref, ln2b_ref,
                  wd_ref, bd_ref, out_ref):
    xb = x_ref[...].reshape(TBLK, D)
    f32 = jnp.float32

    q = jnp.dot(xb, wq_ref[...], preferred_element_type=f32) + bq_ref[...]
    k = jnp.dot(xb, wk_ref[...], preferred_element_type=f32) + bk_ref[...]
    v = jnp.dot(xb, wv_ref[...], preferred_element_type=f32) + bv_ref[...]

    r_id = jax.lax.broadcasted_iota(jnp.int32, (TBLK, TBLK), 0)
    c_id = jax.lax.broadcasted_iota(jnp.int32, (TBLK, TBLK), 1)
    same_seq = (r_id % SBLK) == (c_id % SBLK)

    heads = []
    scale = 1.0 / (DH ** 0.5)
    for h in range(HT):
        qh = q[:, h * DH:(h + 1) * DH]
        kh = k[:, h * DH:(h + 1) * DH]
        vh = v[:, h * DH:(h + 1) * DH]
        s = jax.lax.dot_general(qh, kh, (((1,), (1,)), ((), ())),
                                preferred_element_type=f32) * scale
        s = jnp.where(same_seq, s, -1e9)
        m = jnp.max(s, axis=-1, keepdims=True)
        p = jnp.exp(s - m)
        attn = p / jnp.sum(p, axis=-1, keepdims=True)
        heads.append(jnp.dot(attn, vh, preferred_element_type=f32))
    o = jnp.concatenate(heads, axis=-1)
    o = jnp.dot(o, wo_ref[...], preferred_element_type=f32) + bo_ref[...]
    h1 = _ln(xb + o, ln1g_ref[...], ln1b_ref[...])

    logits = jnp.dot(h1, wr_ref[...], preferred_element_type=f32) + br_ref[...]
    eiota = jax.lax.broadcasted_iota(jnp.int32, (TBLK, E), 1)
    m1 = jnp.max(logits, axis=-1, keepdims=True)
    i1 = jnp.min(jnp.where(logits == m1, eiota, E), axis=-1, keepdims=True)
    lmask = jnp.where(eiota == i1, -jnp.inf, logits)
    m2 = jnp.max(lmask, axis=-1, keepdims=True)
    i2 = jnp.min(jnp.where(lmask == m2, eiota, E), axis=-1, keepdims=True)
    t = jnp.exp(m2 - m1)
    g1 = 1.0 / (1.0 + t)
    g2 = t / (1.0 + t)

    acc = jnp.zeros((TBLK, D), f32)
    for e in range(E):
        w = jnp.where(i1 == e, g1, 0.0) + jnp.where(i2 == e, g2, 0.0)
        he = jnp.maximum(
            jnp.dot(h1, we1_ref[e], preferred_element_type=f32)
            + be1_ref[e:e + 1, :], 0.0)
        oe = jnp.dot(he, we2_ref[e], preferred_element_type=f32) + be2_ref[e:e + 1, :]
        acc = acc + w * oe

    h2 = _ln(h1 + acc, ln2g_ref[...], ln2b_ref[...])

    dz = jnp.dot(h2, wd_ref[...], preferred_element_type=f32) + bd_ref[...]
    dz = jnp.where(dz > 0, dz, jnp.exp(jnp.minimum(dz, 0.0)) - 1.0)
    pooled = jnp.max(dz.reshape(L2, 2, SBLK, D), axis=1)
    out_ref[...] = pooled


def _gat_kernel(tc_ref, adj_ref, wg_ref, asrc_ref, adst_ref, out_ref):
    f32 = jnp.float32
    tc = tc_ref[0]
    adj = adj_ref[0]
    hg = jnp.dot(tc, wg_ref[...], preferred_element_type=f32)
    neg = adj <= 0
    heads = []
    for h in range(HN):
        hgh = hg[:, h * DHN:(h + 1) * DHN]
        av_s = asrc_ref[h:h + 1, :]
        av_d = adst_ref[h:h + 1, :]
        src = jax.lax.dot_general(hgh, av_s, (((1,), (1,)), ((), ())),
                                  preferred_element_type=f32)
        dst = jax.lax.dot_general(av_d, hgh, (((1,), (1,)), ((), ())),
                                  preferred_element_type=f32)
        e = src + dst
        e = jnp.where(e >= 0, e, 0.2 * e)
        e = jnp.where(neg, -1e9, e)
        m = jnp.max(e, axis=-1, keepdims=True)
        p = jnp.exp(e - m)
        alpha = p / jnp.sum(p, axis=-1, keepdims=True)
        heads.append(jnp.dot(alpha, hgh, preferred_element_type=f32))
    go = jnp.concatenate(heads, axis=-1)
    out_ref[0] = jnp.maximum(go, 0.0)


@functools.partial(jax.jit, static_argnums=())
def kernel(x, adj, Wq, Wk, Wv, bq, bk, bv, Wo, bo, ln1_g, ln1_b, Wr, br,
           We1, be1, We2, be2, ln2_g, ln2_b, Wd, bd, Wg, a_src, a_dst):
    f32 = jnp.float32
    xt = x.reshape(SEQ, L, D).transpose(1, 0, 2)

    row = lambda a: a.reshape(1, -1)
    grid1 = SEQ // SBLK
    const = lambda shape: pl.BlockSpec(shape, lambda i: (0,) * len(shape))

    distill = pl.pallas_call(
        _block_kernel,
        grid=(grid1,),
        in_specs=[
            pl.BlockSpec((L, SBLK, D), lambda i: (0, i, 0)),
            const((D, D)), const((D, D)), const((D, D)),
            const((1, D)), const((1, D)), const((1, D)),
            const((D, D)), const((1, D)),
            const((1, D)), const((1, D)),
            const((D, E)), const((1, E)),
            const((E, D, D)), const((E, D)),
            const((E, D, D)), const((E, D)),
            const((1, D)), const((1, D)),
            const((D, D)), const((1, D)),
        ],
        out_specs=pl.BlockSpec((L2, SBLK, D), lambda i: (0, i, 0)),
        out_shape=jax.ShapeDtypeStruct((L2, SEQ, D), f32),
    )(xt, Wq, Wk, Wv, row(bq), row(bk), row(bv), Wo, row(bo),
      row(ln1_g), row(ln1_b), Wr, row(br), We1, be1, We2, be2,
      row(ln2_g), row(ln2_b), Wd, row(bd))

    ngraph = B * L2
    go = pl.pallas_call(
        _gat_kernel,
        grid=(ngraph,),
        in_specs=[
            pl.BlockSpec((1, N, D), lambda g: (g % L2, g // L2, 0)),
            pl.BlockSpec((1, N, N), lambda g: (g % 2, 0, 0)),
            pl.BlockSpec((D, OUT), lambda g: (0, 0)),
            pl.BlockSpec((HN, DHN), lambda g: (0, 0)),
            pl.BlockSpec((HN, DHN), lambda g: (0, 0)),
        ],
        out_specs=pl.BlockSpec((1, N, OUT), lambda g: (g % L2, g // L2, 0)),
        out_shape=jax.ShapeDtypeStruct((L2, SEQ, OUT), f32),
    )(distill, adj, Wg, a_src, a_dst)

    return go.reshape(L2, B, N, OUT).transpose(1, 2, 0, 3)


# speedup vs baseline: 1.3664x; 1.0478x over previous
import functools

import jax
import jax.numpy as jnp
from jax.experimental import pallas as pl

B, N, L, D = 2, 512, 32, 64
E, K = 16, 2
HT, HN = 4, 4
OUT = 64
DH = D // HT
DHN = OUT // HN
L2 = L // 2
SEQ = B * N

SBLK = 8
TBLK = SBLK * L

_NEG = -1e9


def _ln(z, g, b):
    m = jnp.mean(z, axis=-1, keepdims=True)
    v = jnp.mean((z - m) ** 2, axis=-1, keepdims=True)
    return (z - m) / jnp.sqrt(v + 1e-5) * g + b


def _block_kernel(x_ref, wq_ref, wk_ref, wv_ref, bq_ref, bk_ref, bv_ref,
                  wo_ref, bo_ref, ln1g_ref, ln1b_ref, wr_ref, br_ref,
                  w1cat_ref, b1cat_ref, w2stack_ref, be2_ref, emat_ref,
                  ln2g_ref, ln2b_ref, wd_ref, bd_ref, out_ref):
    xb = x_ref[...].reshape(TBLK, D)
    f32 = jnp.float32

    scale = 1.0 / (DH ** 0.5)
    q = (jnp.dot(xb, wq_ref[...], preferred_element_type=f32) + bq_ref[...]) * scale
    k = jnp.dot(xb, wk_ref[...], preferred_element_type=f32) + bk_ref[...]
    v = jnp.dot(xb, wv_ref[...], preferred_element_type=f32) + bv_ref[...]

    r_id = jax.lax.broadcasted_iota(jnp.int32, (TBLK, 1), 0) % SBLK
    c_id = jax.lax.broadcasted_iota(jnp.int32, (1, TBLK), 1) % SBLK
    bias = jnp.where(r_id == c_id, 0.0, _NEG).astype(f32)

    heads = []
    for h in range(HT):
        qh = q[:, h * DH:(h + 1) * DH]
        kh = k[:, h * DH:(h + 1) * DH]
        vh = v[:, h * DH:(h + 1) * DH]
        s = jax.lax.dot_general(qh, kh, (((1,), (1,)), ((), ())),
                                preferred_element_type=f32) + bias
        m = jnp.max(s, axis=-1, keepdims=True)
        p = jnp.exp(s - m)
        inv = 1.0 / jnp.sum(p, axis=-1, keepdims=True)
        heads.append(jnp.dot(p, vh, preferred_element_type=f32) * inv)
    o = jnp.concatenate(heads, axis=-1)
    o = jnp.dot(o, wo_ref[...], preferred_element_type=f32) + bo_ref[...]
    h1 = _ln(xb + o, ln1g_ref[...], ln1b_ref[...])

    logits = jnp.dot(h1, wr_ref[...], preferred_element_type=f32) + br_ref[...]
    eiota = jax.lax.broadcasted_iota(jnp.int32, (TBLK, E), 1)
    m1 = jnp.max(logits, axis=-1, keepdims=True)
    i1 = jnp.min(jnp.where(logits == m1, eiota, E), axis=-1, keepdims=True)
    lmask = jnp.where(eiota == i1, -jnp.inf, logits)
    m2 = jnp.max(lmask, axis=-1, keepdims=True)
    i2 = jnp.min(jnp.where(lmask == m2, eiota, E), axis=-1, keepdims=True)
    t = jnp.exp(m2 - m1)
    g1 = 1.0 / (1.0 + t)
    g2 = t / (1.0 + t)
    w16 = jnp.where(eiota == i1, g1, 0.0) + jnp.where(eiota == i2, g2, 0.0)

    hcat = jnp.maximum(
        jnp.dot(h1, w1cat_ref[...], preferred_element_type=f32)
        + b1cat_ref[...], 0.0)
    wexp = jnp.dot(w16, emat_ref[...], preferred_element_type=f32)
    acc = (jnp.dot(hcat * wexp, w2stack_ref[...], preferred_element_type=f32)
           + jnp.dot(w16, be2_ref[...], preferred_element_type=f32))

    h2 = _ln(h1 + acc, ln2g_ref[...], ln2b_ref[...])

    dz = jnp.dot(h2, wd_ref[...], preferred_element_type=f32) + bd_ref[...]
    dz = jnp.where(dz > 0, dz, jnp.exp(jnp.minimum(dz, 0.0)) - 1.0)
    out_ref[...] = jnp.max(dz.reshape(L2, 2, SBLK, D), axis=1)


def _gat_kernel(tc_ref, adj_ref, wg_ref, asrc_ref, adst_ref, out_ref):
    f32 = jnp.float32
    tc = tc_ref[0]
    abias = jnp.where(adj_ref[0] > 0, 0.0, _NEG).astype(f32)
    hg = jnp.dot(tc, wg_ref[...], preferred_element_type=f32)
    hidx = jax.lax.broadcasted_iota(jnp.int32, (N, OUT), 1) // DHN
    alphas = []
    vstack = []
    for h in range(HN):
        hgh = hg[:, h * DHN:(h + 1) * DHN]
        src = jax.lax.dot_general(hgh, asrc_ref[h:h + 1, :],
                                  (((1,), (1,)), ((), ())),
                                  preferred_element_type=f32)
        dst = jax.lax.dot_general(adst_ref[h:h + 1, :], hgh,
                                  (((1,), (1,)), ((), ())),
                                  preferred_element_type=f32)
        z = src + dst
        e = jnp.maximum(z, 0.2 * z) + abias
        m = jnp.max(e, axis=-1, keepdims=True)
        p = jnp.exp(e - m)
        inv = 1.0 / jnp.sum(p, axis=-1, keepdims=True)
        alphas.append(p * inv)
        vstack.append(jnp.where(hidx == h, hg, 0.0))
    alpha_cat = jnp.concatenate(alphas, axis=-1)
    v_stack = jnp.concatenate(vstack, axis=0)
    go = jnp.dot(alpha_cat, v_stack, preferred_element_type=f32)
    out_ref[0] = jnp.maximum(go, 0.0)


@functools.partial(jax.jit, static_argnums=())
def kernel(x, adj, Wq, Wk, Wv, bq, bk, bv, Wo, bo, ln1_g, ln1_b, Wr, br,
           We1, be1, We2, be2, ln2_g, ln2_b, Wd, bd, Wg, a_src, a_dst):
    f32 = jnp.float32
    xt = x.reshape(SEQ, L, D).transpose(1, 0, 2)
    w1cat = We1.transpose(1, 0, 2).reshape(D, E * D)
    b1cat = be1.reshape(1, E * D)
    w2stack = We2.reshape(E * D, D)
    emat = jnp.kron(jnp.eye(E, dtype=f32), jnp.ones((1, D), f32))

    row = lambda a: a.reshape(1, -1)
    grid1 = SEQ // SBLK
    const = lambda shape: pl.BlockSpec(shape, lambda i: (0,) * len(shape))

    distill = pl.pallas_call(
        _block_kernel,
        grid=(grid1,),
        in_specs=[
            pl.BlockSpec((L, SBLK, D), lambda i: (0, i, 0)),
            const((D, D)), const((D, D)), const((D, D)),
            const((1, D)), const((1, D)), const((1, D)),
            const((D, D)), const((1, D)),
            const((1, D)), const((1, D)),
            const((D, E)), const((1, E)),
            const((D, E * D)), const((1, E * D)),
            const((E * D, D)), const((E, D)),
            const((E, E * D)),
            const((1, D)), const((1, D)),
            const((D, D)), const((1, D)),
        ],
        out_specs=pl.BlockSpec((L2, SBLK, D), lambda i: (0, i, 0)),
        out_shape=jax.ShapeDtypeStruct((L2, SEQ, D), f32),
    )(xt, Wq, Wk, Wv, row(bq), row(bk), row(bv), Wo, row(bo),
      row(ln1_g), row(ln1_b), Wr, row(br), w1cat, b1cat, w2stack, be2,
      emat, row(ln2_g), row(ln2_b), Wd, row(bd))

    ngraph = B * L2
    go = pl.pallas_call(
        _gat_kernel,
        grid=(ngraph,),
        in_specs=[
            pl.BlockSpec((1, N, D), lambda g: (g % L2, g // L2, 0)),
            pl.BlockSpec((1, N, N), lambda g: (g % 2, 0, 0)),
            pl.BlockSpec((D, OUT), lambda g: (0, 0)),
            pl.BlockSpec((HN, DHN), lambda g: (0, 0)),
            pl.BlockSpec((HN, DHN), lambda g: (0, 0)),
        ],
        out_specs=pl.BlockSpec((1, N, OUT), lambda g: (g % L2, g // L2, 0)),
        out_shape=jax.ShapeDtypeStruct((L2, SEQ, OUT), f32),
    )(distill, adj, Wg, a_src, a_dst)

    return go.reshape(L2, B, N, OUT).transpose(1, 2, 0, 3)


# speedup vs baseline: 1.5886x; 1.1626x over previous
import functools

import jax
import jax.numpy as jnp
from jax.experimental import pallas as pl

B, N, L, D = 2, 512, 32, 64
E, K = 16, 2
HT, HN = 4, 4
OUT = 64
DH = D // HT
DHN = OUT // HN
L2 = L // 2
SEQ = B * N

SBLK = 8
TBLK = SBLK * L

_NEG = -1e9


def _ln(z, g, b):
    m = jnp.mean(z, axis=-1, keepdims=True)
    v = jnp.mean((z - m) ** 2, axis=-1, keepdims=True)
    return (z - m) / jnp.sqrt(v + 1e-5) * g + b


def _block_kernel(x_ref, wq_ref, wk_ref, wv_ref, bq_ref, bk_ref, bv_ref,
                  wo_ref, bo_ref, ln1g_ref, ln1b_ref, wr_ref, br_ref,
                  w1cat_ref, b1cat_ref, w2stack_ref, be2_ref, emat_ref,
                  ln2g_ref, ln2b_ref, wd_ref, bd_ref, out_ref):
    xb = x_ref[...].reshape(TBLK, D)
    f32 = jnp.float32

    scale = 1.0 / (DH ** 0.5)
    q = (jnp.dot(xb, wq_ref[...], preferred_element_type=f32) + bq_ref[...]) * scale
    k = jnp.dot(xb, wk_ref[...], preferred_element_type=f32) + bk_ref[...]
    v = jnp.dot(xb, wv_ref[...], preferred_element_type=f32) + bv_ref[...]

    r_id = jax.lax.broadcasted_iota(jnp.int32, (TBLK, 1), 0) % SBLK
    c_id = jax.lax.broadcasted_iota(jnp.int32, (1, TBLK), 1) % SBLK
    bias = jnp.where(r_id == c_id, 0.0, _NEG).astype(f32)

    heads = []
    for h in range(HT):
        qh = q[:, h * DH:(h + 1) * DH]
        kh = k[:, h * DH:(h + 1) * DH]
        vh = v[:, h * DH:(h + 1) * DH]
        s = jax.lax.dot_general(qh, kh, (((1,), (1,)), ((), ())),
                                preferred_element_type=f32) + bias
        p = jnp.exp(s)
        inv = 1.0 / jnp.sum(p, axis=-1, keepdims=True)
        heads.append(jnp.dot(p, vh, preferred_element_type=f32) * inv)
    o = jnp.concatenate(heads, axis=-1)
    o = jnp.dot(o, wo_ref[...], preferred_element_type=f32) + bo_ref[...]
    h1 = _ln(xb + o, ln1g_ref[...], ln1b_ref[...])

    logits = jnp.dot(h1, wr_ref[...], preferred_element_type=f32) + br_ref[...]
    eiota = jax.lax.broadcasted_iota(jnp.int32, (TBLK, E), 1)
    m1 = jnp.max(logits, axis=-1, keepdims=True)
    i1 = jnp.min(jnp.where(logits == m1, eiota, E), axis=-1, keepdims=True)
    lmask = jnp.where(eiota == i1, -jnp.inf, logits)
    m2 = jnp.max(lmask, axis=-1, keepdims=True)
    i2 = jnp.min(jnp.where(lmask == m2, eiota, E), axis=-1, keepdims=True)
    t = jnp.exp(m2 - m1)
    g1 = 1.0 / (1.0 + t)
    g2 = t / (1.0 + t)
    w16 = jnp.where(eiota == i1, g1, 0.0) + jnp.where(eiota == i2, g2, 0.0)

    bf = jnp.bfloat16
    hcat = jnp.maximum(
        jnp.dot(h1.astype(bf), w1cat_ref[...].astype(bf),
                preferred_element_type=f32) + b1cat_ref[...], 0.0)
    wexp = jnp.dot(w16.astype(bf), emat_ref[...].astype(bf),
                   preferred_element_type=f32)
    acc = (jnp.dot((hcat * wexp).astype(bf), w2stack_ref[...].astype(bf),
                   preferred_element_type=f32)
           + jnp.dot(w16, be2_ref[...], preferred_element_type=f32))

    h2 = _ln(h1 + acc, ln2g_ref[...], ln2b_ref[...])

    dz = jnp.dot(h2.astype(bf), wd_ref[...].astype(bf),
                 preferred_element_type=f32) + bd_ref[...]
    dz = jnp.where(dz > 0, dz, jnp.exp(jnp.minimum(dz, 0.0)) - 1.0)
    out_ref[...] = jnp.max(dz.reshape(L2, 2, SBLK, D), axis=1)


def _gat_kernel(tc_ref, adj_ref, wg_ref, asrc_ref, adst_ref, out_ref):
    f32 = jnp.float32
    bf = jnp.bfloat16
    tc = tc_ref[0]
    abias = jnp.where(adj_ref[0] > 0, 0.0, _NEG).astype(f32)
    hg = jnp.dot(tc.astype(bf), wg_ref[...].astype(bf),
                 preferred_element_type=f32)
    hidx = jax.lax.broadcasted_iota(jnp.int32, (N, OUT), 1) // DHN
    alphas = []
    vstack = []
    for h in range(HN):
        hgh = hg[:, h * DHN:(h + 1) * DHN]
        src = jax.lax.dot_general(hgh, asrc_ref[h:h + 1, :],
                                  (((1,), (1,)), ((), ())),
                                  preferred_element_type=f32)
        dst = jax.lax.dot_general(adst_ref[h:h + 1, :], hgh,
                                  (((1,), (1,)), ((), ())),
                                  preferred_element_type=f32)
        z = src + dst
        e = jnp.maximum(z, 0.2 * z) + abias
        p = jnp.exp(e)
        inv = 1.0 / jnp.sum(p, axis=-1, keepdims=True)
        alphas.append(p * inv)
        vstack.append(jnp.where(hidx == h, hg, 0.0))
    alpha_cat = jnp.concatenate(alphas, axis=-1)
    v_stack = jnp.concatenate(vstack, axis=0)
    go = jnp.dot(alpha_cat.astype(bf), v_stack.astype(bf),
                 preferred_element_type=f32)
    out_ref[0] = jnp.maximum(go, 0.0)


@functools.partial(jax.jit, static_argnums=())
def kernel(x, adj, Wq, Wk, Wv, bq, bk, bv, Wo, bo, ln1_g, ln1_b, Wr, br,
           We1, be1, We2, be2, ln2_g, ln2_b, Wd, bd, Wg, a_src, a_dst):
    f32 = jnp.float32
    xt = x.reshape(SEQ, L, D).transpose(1, 0, 2)
    w1cat = We1.transpose(1, 0, 2).reshape(D, E * D)
    b1cat = be1.reshape(1, E * D)
    w2stack = We2.reshape(E * D, D)
    emat = jnp.kron(jnp.eye(E, dtype=f32), jnp.ones((1, D), f32))

    row = lambda a: a.reshape(1, -1)
    grid1 = SEQ // SBLK
    const = lambda shape: pl.BlockSpec(shape, lambda i: (0,) * len(shape))

    distill = pl.pallas_call(
        _block_kernel,
        grid=(grid1,),
        in_specs=[
            pl.BlockSpec((L, SBLK, D), lambda i: (0, i, 0)),
            const((D, D)), const((D, D)), const((D, D)),
            const((1, D)), const((1, D)), const((1, D)),
            const((D, D)), const((1, D)),
            const((1, D)), const((1, D)),
            const((D, E)), const((1, E)),
            const((D, E * D)), const((1, E * D)),
            const((E * D, D)), const((E, D)),
            const((E, E * D)),
            const((1, D)), const((1, D)),
            const((D, D)), const((1, D)),
        ],
        out_specs=pl.BlockSpec((L2, SBLK, D), lambda i: (0, i, 0)),
        out_shape=jax.ShapeDtypeStruct((L2, SEQ, D), f32),
    )(xt, Wq, Wk, Wv, row(bq), row(bk), row(bv), Wo, row(bo),
      row(ln1_g), row(ln1_b), Wr, row(br), w1cat, b1cat, w2stack, be2,
      emat, row(ln2_g), row(ln2_b), Wd, row(bd))

    ngraph = B * L2
    go = pl.pallas_call(
        _gat_kernel,
        grid=(ngraph,),
        in_specs=[
            pl.BlockSpec((1, N, D), lambda g: (g % L2, g // L2, 0)),
            pl.BlockSpec((1, N, N), lambda g: (g % 2, 0, 0)),
            pl.BlockSpec((D, OUT), lambda g: (0, 0)),
            pl.BlockSpec((HN, DHN), lambda g: (0, 0)),
            pl.BlockSpec((HN, DHN), lambda g: (0, 0)),
        ],
        out_specs=pl.BlockSpec((1, N, OUT), lambda g: (g % L2, g // L2, 0)),
        out_shape=jax.ShapeDtypeStruct((L2, SEQ, OUT), f32),
    )(distill, adj, Wg, a_src, a_dst)

    return go.reshape(L2, B, N, OUT).transpose(1, 2, 0, 3)
